# baseline (speedup 1.0000x reference)
import functools

import jax
import jax.numpy as jnp
from jax import lax
from jax.experimental import pallas as pl
from jax.experimental.pallas import tpu as pltpu
from jax.experimental.pallas import tpu_sc as plsc

_NX, _NY, _NZ = 360, 360, 2
_V = _NZ * _NY * _NX
_B = 2
_C = 64
_P = 120000
_PER = _P // _B
_NPAD = 128
_TR = 16896
_VC = 384
_NCK = _TR // _VC
_CB = 1200
_NB = _PER // _CB
_LO_MAX = _V - _TR


def _sc_body(feat, zarr, yarr, xarr, canvas, masks, coords_v,
             inv_v, gath0, gath1, outt0, outt1, mask0, mask1,
             gsem0, gsem1, wsem0, wsem1):
    c = lax.axis_index("c")
    t = lax.axis_index("s")
    iota = lax.iota(jnp.int32, 16)
    lo = pl.multiple_of(jnp.minimum(t * _TR, _LO_MAX), 128)
    gath = (gath0, gath1)
    outt = (outt0, outt1)
    maskb = (mask0, mask1)
    gsem = (gsem0, gsem1)
    wsem = (wsem0, wsem1)

    with jax.named_scope("ph_fill"):
        def fill_body(ii, carry):
            inv_v[pl.ds(ii * 16, 16)] = jnp.full((16,), -1, jnp.int32)
            return carry
        lax.fori_loop(0, _TR // 16, fill_body, 0)

    def blk_body(blk, carry):
      with jax.named_scope("ph_scan"):
        base_p = pl.multiple_of(c * _PER + blk * _CB, 8)
        pltpu.sync_copy(zarr.at[pl.ds(base_p, _CB)], coords_v.at[0])
        pltpu.sync_copy(yarr.at[pl.ds(base_p, _CB)], coords_v.at[1])
        pltpu.sync_copy(xarr.at[pl.ds(base_p, _CB)], coords_v.at[2])

        @plsc.parallel_loop(0, _CB // 16, unroll=2)
        def _(g):
            zc = coords_v[0, pl.ds(g * 16, 16)]
            yc = coords_v[1, pl.ds(g * 16, 16)]
            xc = coords_v[2, pl.ds(g * 16, 16)]
            v = zc * (_NY * _NX) + yc * _NX + xc
            m = (v >= lo) & (v < lo + _TR)
            idx = jnp.clip(v - lo, 0, _TR - 1)
            plsc.store_scatter(inv_v, [idx], base_p + g * 16 + iota, mask=m)
        return carry
    lax.fori_loop(0, _NB, blk_body, 0)

    def fire_gather(kk, b):
        @plsc.parallel_loop(0, _VC, unroll=4)
        def _(vi):
            for q in range(_C // 16):
                gath[b][vi, pl.ds(q * 16, 16)] = jnp.zeros((16,), jnp.float32)
        for j in range(_VC // 128):
            idx = plsc.Indices(
                inv_v.at[pl.ds(kk * _VC + j * 128, 128)], ignored_value=-1)
            pltpu.async_copy(
                feat.at[idx], gath[b].at[pl.ds(j * 128, 128)], gsem[b])

    def drain_gather(b):
        for j in range(_VC // 128):
            idx = plsc.Indices(
                inv_v.at[pl.ds(j * 128, 128)], ignored_value=-1)
            pltpu.make_async_copy(
                feat.at[idx], gath[b].at[pl.ds(j * 128, 128)], gsem[b]).wait()

    def drain_writes(b):
        pltpu.make_async_copy(
            outt[b].at[:, pl.ds(0, _VC)],
            canvas.at[pl.ds(c * _C, _C), pl.ds(0, _VC)],
            wsem[b]).wait()
        pltpu.make_async_copy(
            maskb[b], masks.at[0, pl.ds(0, _VC)], wsem[b]).wait()

    fire_gather(0, 0)

    def pair_body(i, carry):
        for b in range(2):
            kk = i * 2 + b
            v0 = pl.multiple_of(lo + kk * _VC, 128)

            @pl.when(kk + 1 < _NCK)
            def _():
                fire_gather(kk + 1, b ^ 1)
            with jax.named_scope("ph_gwait"):
                drain_gather(b)

                @pl.when(kk >= 2)
                def _():
                    drain_writes(b)

            with jax.named_scope("ph_tr"):
                @plsc.parallel_loop(0, _VC, unroll=4)
                def _(vi):
                    vv = jnp.full((16,), 0, jnp.int32) + vi
                    for q in range(_C // 16):
                        plsc.store_scatter(
                            outt[b], [iota + q * 16, vv],
                            gath[b][vi, pl.ds(q * 16, 16)])

            with jax.named_scope("ph_wr"):
                for o in range(_VC // 16):
                    vv = inv_v[pl.ds(kk * _VC + o * 16, 16)]
                    maskb[b][pl.ds(o * 16, 16)] = jnp.where(
                        vv >= 0, jnp.int32(1), jnp.int32(0))

                pltpu.async_copy(
                    outt[b].at[:, pl.ds(0, _VC)],
                    canvas.at[pl.ds(c * _C, _C), pl.ds(v0, _VC)],
                    wsem[b])
                pltpu.async_copy(
                    maskb[b], masks.at[c, pl.ds(v0, _VC)], wsem[b])
        return carry
    lax.fori_loop(0, _NCK // 2, pair_body, 0)
    drain_writes(0)
    drain_writes(1)


@functools.partial(
    pl.kernel,
    out_type=(
        jax.ShapeDtypeStruct((_B * _C, _V), jnp.float32),
        jax.ShapeDtypeStruct((_B, _V), jnp.int32),
    ),
    mesh=plsc.VectorSubcoreMesh(core_axis_name="c", subcore_axis_name="s"),
    compiler_params=pltpu.CompilerParams(
        needs_layout_passes=False, use_tc_tiling_on_sc=False),
    scratch_types=[
        pltpu.VMEM((3, _CB), jnp.int32),
        pltpu.VMEM((_TR,), jnp.int32),
        pltpu.VMEM((_VC, _C), jnp.float32),
        pltpu.VMEM((_VC, _C), jnp.float32),
        pltpu.VMEM((_C, _VC + 1), jnp.float32),
        pltpu.VMEM((_C, _VC + 1), jnp.float32),
        pltpu.VMEM((_VC,), jnp.int32),
        pltpu.VMEM((_VC,), jnp.int32),
        pltpu.SemaphoreType.DMA,
        pltpu.SemaphoreType.DMA,
        pltpu.SemaphoreType.DMA,
        pltpu.SemaphoreType.DMA,
    ],
)
def _scatter3d_sc(feat, zarr, yarr, xarr, canvas, masks, *scratch):
    _sc_body(feat, zarr, yarr, xarr, canvas, masks, *scratch)


def kernel(pillar_features, coords):
    canvas, masks_i32 = _scatter3d_sc(
        pillar_features, coords[:, 1], coords[:, 2], coords[:, 3])
    batch_spatial_features = canvas.reshape(_B, _C * _NZ, _NY, _NX)
    masks = masks_i32 != 0
    return batch_spatial_features, masks


# speedup vs baseline: 1.0076x; 1.0076x over previous
import functools

import jax
import jax.numpy as jnp
from jax import lax
from jax.experimental import pallas as pl
from jax.experimental.pallas import tpu as pltpu
from jax.experimental.pallas import tpu_sc as plsc

_NX, _NY, _NZ = 360, 360, 2
_V = _NZ * _NY * _NX
_B = 2
_C = 64
_P = 120000
_PER = _P // _B
_NPAD = 128
_TR = 16896
_VC = 384
_NCK = _TR // _VC
_CB = 1200
_NB = _PER // _CB
_LO_MAX = _V - _TR


def _sc_body(feat, zarr, yarr, xarr, canvas, masks, coords_v,
             inv_v, gath0, gath1, outt0, outt1, mask0, mask1,
             gsem0, gsem1, wsem0, wsem1):
    c = lax.axis_index("c")
    t = lax.axis_index("s")
    iota = lax.iota(jnp.int32, 16)
    lo = pl.multiple_of(jnp.minimum(t * _TR, _LO_MAX), 128)
    gath = (gath0, gath1)
    outt = (outt0, outt1)
    maskb = (mask0, mask1)
    gsem = (gsem0, gsem1)
    wsem = (wsem0, wsem1)

    with jax.named_scope("ph_fill"):
        def fill_body(ii, carry):
            inv_v[pl.ds(ii * 16, 16)] = jnp.full((16,), -1, jnp.int32)
            return carry
        lax.fori_loop(0, _TR // 16, fill_body, 0)

    def blk_body(blk, carry):
      with jax.named_scope("ph_scan"):
        base_p = pl.multiple_of(c * _PER + blk * _CB, 8)
        pltpu.sync_copy(zarr.at[pl.ds(base_p, _CB)], coords_v.at[0])
        pltpu.sync_copy(yarr.at[pl.ds(base_p, _CB)], coords_v.at[1])
        pltpu.sync_copy(xarr.at[pl.ds(base_p, _CB)], coords_v.at[2])

        @plsc.parallel_loop(0, _CB // 16, unroll=2)
        def _(g):
            zc = coords_v[0, pl.ds(g * 16, 16)]
            yc = coords_v[1, pl.ds(g * 16, 16)]
            xc = coords_v[2, pl.ds(g * 16, 16)]
            v = zc * (_NY * _NX) + yc * _NX + xc
            m = (v >= lo) & (v < lo + _TR)
            idx = jnp.clip(v - lo, 0, _TR - 1)
            plsc.store_scatter(inv_v, [idx], base_p + g * 16 + iota, mask=m)
        return carry
    lax.fori_loop(0, _NB, blk_body, 0)

    def fire_gather(kk, b):
        @plsc.parallel_loop(0, _VC, unroll=4)
        def _(vi):
            for q in range(_C // 16):
                gath[b][vi, pl.ds(q * 16, 16)] = jnp.zeros((16,), jnp.float32)
        for j in range(_VC // 128):
            idx = plsc.Indices(
                inv_v.at[pl.ds(kk * _VC + j * 128, 128)], ignored_value=-1)
            pltpu.async_copy(
                feat.at[idx], gath[b].at[pl.ds(j * 128, 128)], gsem[b])

    def drain_gather(b):
        for j in range(_VC // 128):
            idx = plsc.Indices(
                inv_v.at[pl.ds(j * 128, 128)], ignored_value=-1)
            pltpu.make_async_copy(
                feat.at[idx], gath[b].at[pl.ds(j * 128, 128)], gsem[b]).wait()

    def drain_writes(b):
        pltpu.make_async_copy(
            outt[b].at[:, pl.ds(0, _VC)],
            canvas.at[pl.ds(c * _C, _C), pl.ds(0, _VC)],
            wsem[b]).wait()
        pltpu.make_async_copy(
            maskb[b], masks.at[pl.ds(0, _VC)], wsem[b]).wait()

    fire_gather(0, 0)

    def pair_body(i, carry):
        for b in range(2):
            kk = i * 2 + b
            v0 = pl.multiple_of(lo + kk * _VC, 128)

            @pl.when(kk + 1 < _NCK)
            def _():
                fire_gather(kk + 1, b ^ 1)
            with jax.named_scope("ph_gwait"):
                drain_gather(b)

                @pl.when(kk >= 2)
                def _():
                    drain_writes(b)

            with jax.named_scope("ph_tr"):
                @plsc.parallel_loop(0, _VC, unroll=4)
                def _(vi):
                    vv = jnp.full((16,), 0, jnp.int32) + vi
                    for q in range(_C // 16):
                        plsc.store_scatter(
                            outt[b], [iota + q * 16, vv],
                            gath[b][vi, pl.ds(q * 16, 16)])

            with jax.named_scope("ph_wr"):
                for o in range(_VC // 16):
                    vv = inv_v[pl.ds(kk * _VC + o * 16, 16)]
                    maskb[b][pl.ds(o * 16, 16)] = jnp.where(
                        vv >= 0, jnp.int32(1), jnp.int32(0))

                pltpu.async_copy(
                    outt[b].at[:, pl.ds(0, _VC)],
                    canvas.at[pl.ds(c * _C, _C), pl.ds(v0, _VC)],
                    wsem[b])
                pltpu.async_copy(
                    maskb[b], masks.at[pl.ds(c * _V + v0, _VC)], wsem[b])
        return carry
    lax.fori_loop(0, _NCK // 2, pair_body, 0)
    drain_writes(0)
    drain_writes(1)


@functools.partial(
    pl.kernel,
    out_type=(
        jax.ShapeDtypeStruct((_B * _C, _V), jnp.float32),
        jax.ShapeDtypeStruct((_B * _V,), jnp.int32),
    ),
    mesh=plsc.VectorSubcoreMesh(core_axis_name="c", subcore_axis_name="s"),
    compiler_params=pltpu.CompilerParams(
        needs_layout_passes=False, use_tc_tiling_on_sc=False),
    scratch_types=[
        pltpu.VMEM((3, _CB), jnp.int32),
        pltpu.VMEM((_TR,), jnp.int32),
        pltpu.VMEM((_VC, _C), jnp.float32),
        pltpu.VMEM((_VC, _C), jnp.float32),
        pltpu.VMEM((_C, _VC + 1), jnp.float32),
        pltpu.VMEM((_C, _VC + 1), jnp.float32),
        pltpu.VMEM((_VC,), jnp.int32),
        pltpu.VMEM((_VC,), jnp.int32),
        pltpu.SemaphoreType.DMA,
        pltpu.SemaphoreType.DMA,
        pltpu.SemaphoreType.DMA,
        pltpu.SemaphoreType.DMA,
    ],
)
def _scatter3d_sc(feat, zarr, yarr, xarr, canvas, masks, *scratch):
    _sc_body(feat, zarr, yarr, xarr, canvas, masks, *scratch)


def kernel(pillar_features, coords):
    canvas, masks_i32 = _scatter3d_sc(
        pillar_features, coords[:, 1], coords[:, 2], coords[:, 3])
    batch_spatial_features = canvas.reshape(_B, _C * _NZ, _NY, _NX)
    masks = masks_i32.reshape(_B, _V) != 0
    return batch_spatial_features, masks


# speedup vs baseline: 1.1060x; 1.0976x over previous
import functools

import jax
import jax.numpy as jnp
from jax import lax
from jax.experimental import pallas as pl
from jax.experimental.pallas import tpu as pltpu
from jax.experimental.pallas import tpu_sc as plsc

_NX, _NY, _NZ = 360, 360, 2
_V = _NZ * _NY * _NX
_B = 2
_C = 64
_P = 120000
_PER = _P // _B
_NPAD = 128
_TR = 16896
_VC = 384
_NCK = _TR // _VC
_CB = 1200
_NB = _PER // _CB
_LO_MAX = _V - _TR


def _sc_body(feat, coords, canvas, masks, coords_v,
             inv_v, gath0, gath1, outt0, outt1, mask0, mask1,
             gsem0, gsem1, wsem0, wsem1):
    c = lax.axis_index("c")
    t = lax.axis_index("s")
    iota = lax.iota(jnp.int32, 16)
    lo = pl.multiple_of(jnp.minimum(t * _TR, _LO_MAX), 128)
    gath = (gath0, gath1)
    outt = (outt0, outt1)
    maskb = (mask0, mask1)
    gsem = (gsem0, gsem1)
    wsem = (wsem0, wsem1)

    with jax.named_scope("ph_fill"):
        def fill_body(ii, carry):
            inv_v[pl.ds(ii * 16, 16)] = jnp.full((16,), -1, jnp.int32)
            return carry
        lax.fori_loop(0, _TR // 16, fill_body, 0)

    def blk_body(blk, carry):
      with jax.named_scope("ph_scan"):
        base_p = pl.multiple_of(c * _PER + blk * _CB, 8)
        pltpu.sync_copy(coords.at[:, pl.ds(base_p, _CB)], coords_v)

        @plsc.parallel_loop(0, _CB // 16, unroll=2)
        def _(g):
            zc = coords_v[0, pl.ds(g * 16, 16)]
            yc = coords_v[1, pl.ds(g * 16, 16)]
            xc = coords_v[2, pl.ds(g * 16, 16)]
            v = zc * (_NY * _NX) + yc * _NX + xc
            m = (v >= lo) & (v < lo + _TR)
            idx = jnp.clip(v - lo, 0, _TR - 1)
            plsc.store_scatter(inv_v, [idx], base_p + g * 16 + iota, mask=m)
        return carry
    lax.fori_loop(0, _NB, blk_body, 0)

    def fire_gather(kk, b):
        @plsc.parallel_loop(0, _VC, unroll=4)
        def _(vi):
            for q in range(_C // 16):
                gath[b][vi, pl.ds(q * 16, 16)] = jnp.zeros((16,), jnp.float32)
        for j in range(_VC // 128):
            idx = plsc.Indices(
                inv_v.at[pl.ds(kk * _VC + j * 128, 128)], ignored_value=-1)
            pltpu.async_copy(
                feat.at[idx], gath[b].at[pl.ds(j * 128, 128)], gsem[b])

    def drain_gather(b):
        for j in range(_VC // 128):
            idx = plsc.Indices(
                inv_v.at[pl.ds(j * 128, 128)], ignored_value=-1)
            pltpu.make_async_copy(
                feat.at[idx], gath[b].at[pl.ds(j * 128, 128)], gsem[b]).wait()

    def drain_writes(b):
        pltpu.make_async_copy(
            outt[b].at[:, pl.ds(0, _VC)],
            canvas.at[pl.ds(c * _C, _C), pl.ds(0, _VC)],
            wsem[b]).wait()
        pltpu.make_async_copy(
            maskb[b], masks.at[pl.ds(0, _VC)], wsem[b]).wait()

    fire_gather(0, 0)

    def pair_body(i, carry):
        for b in range(2):
            kk = i * 2 + b
            v0 = pl.multiple_of(lo + kk * _VC, 128)

            @pl.when(kk + 1 < _NCK)
            def _():
                fire_gather(kk + 1, b ^ 1)
            with jax.named_scope("ph_gwait"):
                drain_gather(b)

                @pl.when(kk >= 2)
                def _():
                    drain_writes(b)

            with jax.named_scope("ph_tr"):
                @plsc.parallel_loop(0, _VC, unroll=4)
                def _(vi):
                    vv = jnp.full((16,), 0, jnp.int32) + vi
                    for q in range(_C // 16):
                        plsc.store_scatter(
                            outt[b], [iota + q * 16, vv],
                            gath[b][vi, pl.ds(q * 16, 16)])

            with jax.named_scope("ph_wr"):
                for o in range(_VC // 16):
                    vv = inv_v[pl.ds(kk * _VC + o * 16, 16)]
                    maskb[b][pl.ds(o * 16, 16)] = jnp.where(
                        vv >= 0, jnp.int32(1), jnp.int32(0))

                pltpu.async_copy(
                    outt[b].at[:, pl.ds(0, _VC)],
                    canvas.at[pl.ds(c * _C, _C), pl.ds(v0, _VC)],
                    wsem[b])
                pltpu.async_copy(
                    maskb[b], masks.at[pl.ds(c * _V + v0, _VC)], wsem[b])
        return carry
    lax.fori_loop(0, _NCK // 2, pair_body, 0)
    drain_writes(0)
    drain_writes(1)


@functools.partial(
    pl.kernel,
    out_type=(
        jax.ShapeDtypeStruct((_B * _C, _V), jnp.float32),
        jax.ShapeDtypeStruct((_B * _V,), jnp.int32),
    ),
    mesh=plsc.VectorSubcoreMesh(core_axis_name="c", subcore_axis_name="s"),
    compiler_params=pltpu.CompilerParams(
        needs_layout_passes=False, use_tc_tiling_on_sc=False),
    scratch_types=[
        pltpu.VMEM((3, _CB), jnp.int32),
        pltpu.VMEM((_TR,), jnp.int32),
        pltpu.VMEM((_VC, _C), jnp.float32),
        pltpu.VMEM((_VC, _C), jnp.float32),
        pltpu.VMEM((_C, _VC + 1), jnp.float32),
        pltpu.VMEM((_C, _VC + 1), jnp.float32),
        pltpu.VMEM((_VC,), jnp.int32),
        pltpu.VMEM((_VC,), jnp.int32),
        pltpu.SemaphoreType.DMA,
        pltpu.SemaphoreType.DMA,
        pltpu.SemaphoreType.DMA,
        pltpu.SemaphoreType.DMA,
    ],
)
def _scatter3d_sc(feat, coords, canvas, masks, *scratch):
    _sc_body(feat, coords, canvas, masks, *scratch)


def kernel(pillar_features, coords):
    zyx = coords[:, 1:4].T
    canvas, masks_i32 = _scatter3d_sc(pillar_features, zyx)
    batch_spatial_features = canvas.reshape(_B, _C * _NZ, _NY, _NX)
    masks = masks_i32.reshape(_B, _V) != 0
    return batch_spatial_features, masks


# speedup vs baseline: 1.1090x; 1.0028x over previous
import functools

import jax
import jax.numpy as jnp
from jax import lax
from jax.experimental import pallas as pl
from jax.experimental.pallas import tpu as pltpu
from jax.experimental.pallas import tpu_sc as plsc

_NX, _NY, _NZ = 360, 360, 2
_V = _NZ * _NY * _NX
_B = 2
_C = 64
_P = 120000
_PER = _P // _B
_NPAD = 128
_TR = 16896
_VC = 384
_NCK = _TR // _VC
_CB = 1200
_NB = _PER // _CB
_LO_MAX = _V - _TR


def _sc_body(feat, coords, canvas, masks, coords_v,
             inv_v, gath0, gath1, outt0, outt1, mask0, mask1,
             gsem0, gsem1, wsem0, wsem1):
    c = lax.axis_index("c")
    t = lax.axis_index("s")
    iota = lax.iota(jnp.int32, 16)
    lo = pl.multiple_of(jnp.minimum(t * _TR, _LO_MAX), 128)
    gath = (gath0, gath1)
    outt = (outt0, outt1)
    maskb = (mask0, mask1)
    gsem = (gsem0, gsem1)
    wsem = (wsem0, wsem1)

    def fill_body(ii, carry):
        inv_v[pl.ds(ii * 16, 16)] = jnp.full((16,), -1, jnp.int32)
        return carry
    lax.fori_loop(0, _TR // 16, fill_body, 0)

    def blk_body(blk, carry):
        base_p = pl.multiple_of(c * _PER + blk * _CB, 8)
        pltpu.sync_copy(coords.at[:, pl.ds(base_p, _CB)], coords_v)

        @plsc.parallel_loop(0, _CB // 16, unroll=2)
        def _(g):
            zc = coords_v[0, pl.ds(g * 16, 16)]
            yc = coords_v[1, pl.ds(g * 16, 16)]
            xc = coords_v[2, pl.ds(g * 16, 16)]
            v = zc * (_NY * _NX) + yc * _NX + xc
            m = (v >= lo) & (v < lo + _TR)
            idx = jnp.clip(v - lo, 0, _TR - 1)
            plsc.store_scatter(inv_v, [idx], base_p + g * 16 + iota, mask=m)
        return carry
    lax.fori_loop(0, _NB, blk_body, 0)

    def fire_gather(kk, b):
        @plsc.parallel_loop(0, _VC, unroll=4)
        def _(vi):
            for q in range(_C // 16):
                gath[b][vi, pl.ds(q * 16, 16)] = jnp.zeros((16,), jnp.float32)
        for j in range(_VC // 128):
            idx = plsc.Indices(
                inv_v.at[pl.ds(kk * _VC + j * 128, 128)], ignored_value=-1)
            pltpu.async_copy(
                feat.at[idx], gath[b].at[pl.ds(j * 128, 128)], gsem[b])

    def drain_gather(b):
        for j in range(_VC // 128):
            idx = plsc.Indices(
                inv_v.at[pl.ds(j * 128, 128)], ignored_value=-1)
            pltpu.make_async_copy(
                feat.at[idx], gath[b].at[pl.ds(j * 128, 128)], gsem[b]).wait()

    def drain_writes(b):
        pltpu.make_async_copy(
            outt[b].at[:, pl.ds(0, _VC)],
            canvas.at[pl.ds(c * _C, _C), pl.ds(0, _VC)],
            wsem[b]).wait()
        pltpu.make_async_copy(
            maskb[b], masks.at[pl.ds(0, _VC)], wsem[b]).wait()

    fire_gather(0, 0)

    def pair_body(i, carry):
        for b in range(2):
            kk = i * 2 + b
            v0 = pl.multiple_of(lo + kk * _VC, 128)

            @pl.when(kk + 1 < _NCK)
            def _():
                fire_gather(kk + 1, b ^ 1)
            drain_gather(b)

            @pl.when(kk >= 2)
            def _():
                drain_writes(b)

            @plsc.parallel_loop(0, _VC, unroll=4)
            def _(vi):
                vv = jnp.full((16,), 0, jnp.int32) + vi
                for q in range(_C // 16):
                    plsc.store_scatter(
                        outt[b], [iota + q * 16, vv],
                        gath[b][vi, pl.ds(q * 16, 16)])

            for o in range(_VC // 16):
                vv = inv_v[pl.ds(kk * _VC + o * 16, 16)]
                maskb[b][pl.ds(o * 16, 16)] = jnp.where(
                    vv >= 0, jnp.int32(1), jnp.int32(0))

            pltpu.async_copy(
                outt[b].at[:, pl.ds(0, _VC)],
                canvas.at[pl.ds(c * _C, _C), pl.ds(v0, _VC)],
                wsem[b])
            pltpu.async_copy(
                maskb[b], masks.at[pl.ds(c * _V + v0, _VC)], wsem[b])
        return carry
    lax.fori_loop(0, _NCK // 2, pair_body, 0)
    drain_writes(0)
    drain_writes(1)


@functools.partial(
    pl.kernel,
    out_type=(
        jax.ShapeDtypeStruct((_B * _C, _V), jnp.float32),
        jax.ShapeDtypeStruct((_B * _V,), jnp.int32),
    ),
    mesh=plsc.VectorSubcoreMesh(core_axis_name="c", subcore_axis_name="s"),
    compiler_params=pltpu.CompilerParams(
        needs_layout_passes=False, use_tc_tiling_on_sc=False),
    scratch_types=[
        pltpu.VMEM((3, _CB), jnp.int32),
        pltpu.VMEM((_TR,), jnp.int32),
        pltpu.VMEM((_VC, _C), jnp.float32),
        pltpu.VMEM((_VC, _C), jnp.float32),
        pltpu.VMEM((_C, _VC + 1), jnp.float32),
        pltpu.VMEM((_C, _VC + 1), jnp.float32),
        pltpu.VMEM((_VC,), jnp.int32),
        pltpu.VMEM((_VC,), jnp.int32),
        pltpu.SemaphoreType.DMA,
        pltpu.SemaphoreType.DMA,
        pltpu.SemaphoreType.DMA,
        pltpu.SemaphoreType.DMA,
    ],
)
def _scatter3d_sc(feat, coords, canvas, masks, *scratch):
    _sc_body(feat, coords, canvas, masks, *scratch)


def kernel(pillar_features, coords):
    zyx = coords[:, 1:4].T
    canvas, masks_i32 = _scatter3d_sc(pillar_features, zyx)
    batch_spatial_features = canvas.reshape(_B, _C * _NZ, _NY, _NX)
    masks = masks_i32.reshape(_B, _V) != 0
    return batch_spatial_features, masks


# speedup vs baseline: 1.1543x; 1.0409x over previous
import functools

import jax
import jax.numpy as jnp
from jax import lax
from jax.experimental import pallas as pl
from jax.experimental.pallas import tpu as pltpu
from jax.experimental.pallas import tpu_sc as plsc

_NX, _NY, _NZ = 360, 360, 2
_V = _NZ * _NY * _NX
_B = 2
_C = 64
_P = 120000
_PER = _P // _B
_NPAD = 128
_TR = 16896
_VC = 384
_NCK = _TR // _VC
_CB = 1200
_NB = _PER // _CB
_LO_MAX = _V - _TR


def _sc_body(feat, coords, canvas, masks, coords_v0, coords_v1,
             inv_v, gath0, gath1, outt0, outt1, mask0, mask1,
             gsem0, gsem1, wsem0, wsem1):
    c = lax.axis_index("c")
    t = lax.axis_index("s")
    iota = lax.iota(jnp.int32, 16)
    lo = pl.multiple_of(jnp.minimum(t * _TR, _LO_MAX), 128)
    cbuf = (coords_v0, coords_v1)
    gath = (gath0, gath1)
    outt = (outt0, outt1)
    maskb = (mask0, mask1)
    gsem = (gsem0, gsem1)
    wsem = (wsem0, wsem1)

    def fill_body(ii, carry):
        inv_v[pl.ds(ii * 16, 16)] = jnp.full((16,), -1, jnp.int32)
        return carry
    lax.fori_loop(0, _TR // 16, fill_body, 0)

    def fire_coords(blk, b):
        base_p = pl.multiple_of(c * _PER + blk * _CB, 8)
        pltpu.async_copy(coords.at[:, pl.ds(base_p, _CB)], cbuf[b], gsem[b])

    def drain_coords(b):
        pltpu.make_async_copy(
            coords.at[:, pl.ds(0, _CB)], cbuf[b], gsem[b]).wait()

    fire_coords(0, 0)

    def blk_pair(i, carry):
        for b in range(2):
            blk = i * 2 + b
            base_p = pl.multiple_of(c * _PER + blk * _CB, 8)

            @pl.when(blk + 1 < _NB)
            def _():
                fire_coords(blk + 1, b ^ 1)
            drain_coords(b)

            @plsc.parallel_loop(0, _CB // 16, unroll=2)
            def _(g):
                zc = cbuf[b][0, pl.ds(g * 16, 16)]
                yc = cbuf[b][1, pl.ds(g * 16, 16)]
                xc = cbuf[b][2, pl.ds(g * 16, 16)]
                v = zc * (_NY * _NX) + yc * _NX + xc
                m = (v >= lo) & (v < lo + _TR)
                idx = jnp.clip(v - lo, 0, _TR - 1)
                plsc.store_scatter(inv_v, [idx], base_p + g * 16 + iota,
                                   mask=m)
        return carry
    lax.fori_loop(0, _NB // 2, blk_pair, 0)

    def fire_gather(kk, b):
        @plsc.parallel_loop(0, _VC, unroll=4)
        def _(vi):
            for q in range(_C // 16):
                gath[b][vi, pl.ds(q * 16, 16)] = jnp.zeros((16,), jnp.float32)
        for j in range(_VC // 128):
            idx = plsc.Indices(
                inv_v.at[pl.ds(kk * _VC + j * 128, 128)], ignored_value=-1)
            pltpu.async_copy(
                feat.at[idx], gath[b].at[pl.ds(j * 128, 128)], gsem[b])

    def drain_gather(b):
        for j in range(_VC // 128):
            idx = plsc.Indices(
                inv_v.at[pl.ds(j * 128, 128)], ignored_value=-1)
            pltpu.make_async_copy(
                feat.at[idx], gath[b].at[pl.ds(j * 128, 128)], gsem[b]).wait()

    def drain_writes(b):
        pltpu.make_async_copy(
            outt[b].at[:, pl.ds(0, _VC)],
            canvas.at[pl.ds(c * _C, _C), pl.ds(0, _VC)],
            wsem[b]).wait()
        pltpu.make_async_copy(
            maskb[b], masks.at[pl.ds(0, _VC)], wsem[b]).wait()

    fire_gather(0, 0)

    def pair_body(i, carry):
        for b in range(2):
            kk = i * 2 + b
            v0 = pl.multiple_of(lo + kk * _VC, 128)

            @pl.when(kk + 1 < _NCK)
            def _():
                fire_gather(kk + 1, b ^ 1)
            drain_gather(b)

            @pl.when(kk >= 2)
            def _():
                drain_writes(b)

            @plsc.parallel_loop(0, _VC, unroll=4)
            def _(vi):
                vv = jnp.full((16,), 0, jnp.int32) + vi
                for q in range(_C // 16):
                    plsc.store_scatter(
                        outt[b], [iota + q * 16, vv],
                        gath[b][vi, pl.ds(q * 16, 16)])

            for o in range(_VC // 16):
                vv = inv_v[pl.ds(kk * _VC + o * 16, 16)]
                maskb[b][pl.ds(o * 16, 16)] = jnp.where(
                    vv >= 0, jnp.int32(1), jnp.int32(0))

            pltpu.async_copy(
                outt[b].at[:, pl.ds(0, _VC)],
                canvas.at[pl.ds(c * _C, _C), pl.ds(v0, _VC)],
                wsem[b])
            pltpu.async_copy(
                maskb[b], masks.at[pl.ds(c * _V + v0, _VC)], wsem[b])
        return carry
    lax.fori_loop(0, _NCK // 2, pair_body, 0)
    drain_writes(0)
    drain_writes(1)


@functools.partial(
    pl.kernel,
    out_type=(
        jax.ShapeDtypeStruct((_B * _C, _V), jnp.float32),
        jax.ShapeDtypeStruct((_B * _V,), jnp.int32),
    ),
    mesh=plsc.VectorSubcoreMesh(core_axis_name="c", subcore_axis_name="s"),
    compiler_params=pltpu.CompilerParams(
        needs_layout_passes=False, use_tc_tiling_on_sc=False),
    scratch_types=[
        pltpu.VMEM((3, _CB), jnp.int32),
        pltpu.VMEM((3, _CB), jnp.int32),
        pltpu.VMEM((_TR,), jnp.int32),
        pltpu.VMEM((_VC, _C), jnp.float32),
        pltpu.VMEM((_VC, _C), jnp.float32),
        pltpu.VMEM((_C, _VC + 1), jnp.float32),
        pltpu.VMEM((_C, _VC + 1), jnp.float32),
        pltpu.VMEM((_VC,), jnp.int32),
        pltpu.VMEM((_VC,), jnp.int32),
        pltpu.SemaphoreType.DMA,
        pltpu.SemaphoreType.DMA,
        pltpu.SemaphoreType.DMA,
        pltpu.SemaphoreType.DMA,
    ],
)
def _scatter3d_sc(feat, coords, canvas, masks, *scratch):
    _sc_body(feat, coords, canvas, masks, *scratch)


def kernel(pillar_features, coords):
    zyx = coords[:, 1:4].T
    canvas, masks_i32 = _scatter3d_sc(pillar_features, zyx)
    batch_spatial_features = canvas.reshape(_B, _C * _NZ, _NY, _NX)
    masks = masks_i32.reshape(_B, _V) != 0
    return batch_spatial_features, masks
